# baseline (speedup 1.0000x reference)
import jax
import jax.numpy as jnp
from jax import lax
from jax.experimental import pallas as pl
from jax.experimental.pallas import tpu as pltpu
from jax.experimental.pallas import tpu_sc as plsc

N = 10000
E = 320000
NH = 128
NOUT = 128

NC = 2
NS = 16
NW = NC * NS
EPAD = 327680
EPW = EPAD // NW
CD = 80
NCHD = EPW // CD
CH = 40
NG = EPW // CH
NB = NG // 8
NACC = 10112
RPT = NACC // NS
DUMMY = N
BLK = 2048
GRID = 5

_f32 = jnp.float32
_i32 = jnp.int32


def _mesh():
    return plsc.VectorSubcoreMesh(core_axis_name="c", subcore_axis_name="s")


def _deg_body(row_hbm, col_hbm, hist_hbm, re_hbm, rowb, colb, reb, hist):
    c = lax.axis_index("c")
    s = lax.axis_index("s")
    wid = c * NS + s
    pltpu.sync_copy(row_hbm.at[wid], rowb)
    pltpu.sync_copy(col_hbm.at[wid], colb)

    def zero(i, _):
        hist[pl.ds(i * 16, 16)] = jnp.zeros((16,), _f32)
        return 0

    lax.fori_loop(0, NACC // 16, zero, 0)

    ones = jnp.ones((16,), _f32)

    def outer(i, _):
        def inner(k, _):
            rv = rowb[i, pl.ds(k * 16, 16)]
            cv = colb[i, pl.ds(k * 16, 16)]
            m = rv != cv
            plsc.addupdate_scatter(hist, [cv], ones, mask=m)
            reb[i, pl.ds(k * 16, 16)] = jnp.where(m, rv, DUMMY)
            return 0

        lax.fori_loop(0, CD // 16, inner, 0)
        return 0

    lax.fori_loop(0, NCHD, outer, 0)
    pltpu.sync_copy(hist, hist_hbm.at[pl.ds(wid * NACC, NACC)])
    pltpu.sync_copy(reb, re_hbm.at[wid])


def _deg_call(row3d, col3d):
    kern = pl.kernel(
        _deg_body,
        out_type=[
            jax.ShapeDtypeStruct((NW * NACC,), _f32),
            jax.ShapeDtypeStruct((NW, NCHD, CD), _i32),
        ],
        mesh=_mesh(),
        scratch_types=[
            pltpu.VMEM((NCHD, CD), _i32),
            pltpu.VMEM((NCHD, CD), _i32),
            pltpu.VMEM((NCHD, CD), _i32),
            pltpu.VMEM((NACC,), _f32),
        ],
        compiler_params=pltpu.CompilerParams(needs_layout_passes=False),
    )
    return kern(row3d, col3d)


def _spmm_body(z_hbm, col_hbm, re_hbm, out_hbm, colring, rering, dbuf, acc,
               *sems):
    gsems = sems[0:8]
    ssems = sems[8:16]
    icsems = sems[16:20]
    irsems = sems[20:24]
    c = lax.axis_index("c")
    s = lax.axis_index("s")
    wid = c * NS + s

    slots = [dbuf.at[pl.ds(j * CH, CH)] for j in range(8)]

    def ifire(b, k):
        pltpu.async_copy(col_hbm.at[wid, pl.ds(8 * b, 8)],
                         colring.at[pl.ds(k * 8, 8)], icsems[k])
        pltpu.async_copy(re_hbm.at[wid, pl.ds(8 * b, 8)],
                         rering.at[pl.ds(k * 8, 8)], irsems[k])

    def iwait(b, k):
        pltpu.make_async_copy(col_hbm.at[wid, pl.ds(8 * b, 8)],
                              colring.at[pl.ds(k * 8, 8)],
                              icsems[k]).wait()
        pltpu.make_async_copy(re_hbm.at[wid, pl.ds(8 * b, 8)],
                              rering.at[pl.ds(k * 8, 8)],
                              irsems[k]).wait()

    def gfire(r, j):
        pltpu.async_copy(z_hbm.at[colring.at[r]], slots[j], gsems[j])

    def gwait(r, j):
        pltpu.make_async_copy(z_hbm.at[colring.at[r]], slots[j],
                              gsems[j]).wait()

    def sfire(r, j):
        pltpu.async_copy(slots[j], acc.at[rering.at[r]], ssems[j], add=True)

    def swait(r, j):
        pltpu.make_async_copy(slots[j], acc.at[rering.at[r]],
                              ssems[j]).wait()

    def zrow(i, _):
        for k in range(8):
            dbuf[i, pl.ds(k * 16, 16)] = jnp.zeros((16,), _f32)
        return 0

    lax.fori_loop(0, CH, zrow, 0)

    def zacc(j, _):
        pltpu.sync_copy(slots[0], acc.at[pl.ds(s * RPT + j * CH, CH)])
        return 0

    lax.fori_loop(0, RPT // CH, zacc, 0)
    pltpu.sync_copy(dbuf.at[pl.ds(0, RPT - (RPT // CH) * CH)],
                    acc.at[pl.ds(s * RPT + (RPT // CH) * CH,
                                 RPT - (RPT // CH) * CH)])
    plsc.subcore_barrier()

    ifire(0, 0)
    ifire(1, 1)
    ifire(2, 2)
    iwait(0, 0)
    for j in range(5):
        gfire(j, j)

    def body(p, _):
        pe = lax.rem(p, 2) == 0

        def row(g):
            return lax.rem(g, 32)

        for j in range(16):
            g = 16 * p + j
            gwait(row(g), j % 8)
            sfire(row(g), j % 8)
            if j < 3:
                @pl.when(p > 0)
                def _():
                    swait(row(g - 3), (j - 3) % 8)
            else:
                swait(row(g - 3), (j - 3) % 8)
            if j == 3:
                @pl.when(pe)
                def _():
                    iwait(2 * p + 1, 1)

                @pl.when(jnp.logical_not(pe))
                def _():
                    iwait(2 * p + 1, 3)

                @pl.when(jnp.logical_and(pe, 2 * p + 3 < NB))
                def _():
                    ifire(2 * p + 3, 3)

                @pl.when(jnp.logical_and(jnp.logical_not(pe),
                                         2 * p + 3 < NB))
                def _():
                    ifire(2 * p + 3, 1)
            if j == 11:
                @pl.when(jnp.logical_and(pe, 2 * p + 2 < NB))
                def _():
                    iwait(2 * p + 2, 2)

                @pl.when(jnp.logical_and(jnp.logical_not(pe),
                                         2 * p + 2 < NB))
                def _():
                    iwait(2 * p + 2, 0)

                @pl.when(jnp.logical_and(pe, 2 * p + 4 < NB))
                def _():
                    ifire(2 * p + 4, 0)

                @pl.when(jnp.logical_and(jnp.logical_not(pe),
                                         2 * p + 4 < NB))
                def _():
                    ifire(2 * p + 4, 2)
            if j < 11:
                gfire(row(g + 5), (j + 5) % 8)
            else:
                @pl.when(g + 5 < NG)
                def _():
                    gfire(row(g + 5), (j + 5) % 8)
        return 0

    lax.fori_loop(0, NG // 16, body, 0)
    swait(29, 5)
    swait(30, 6)
    swait(31, 7)

    plsc.subcore_barrier()
    pltpu.sync_copy(acc.at[pl.ds(s * RPT, RPT)],
                    out_hbm.at[c, pl.ds(s * RPT, RPT)])


def _spmm_call(z, col3s, re3s):
    kern = pl.kernel(
        _spmm_body,
        out_type=jax.ShapeDtypeStruct((NC, NACC, NH), _f32),
        mesh=_mesh(),
        scratch_types=[
            pltpu.VMEM((32, CH), _i32),
            pltpu.VMEM((32, CH), _i32),
            pltpu.VMEM((8 * CH, NH), _f32),
            pltpu.VMEM_SHARED((NACC, NH), _f32),
        ] + [pltpu.SemaphoreType.DMA] * 24,
        compiler_params=pltpu.CompilerParams(needs_layout_passes=False),
    )
    return kern(z, col3s, re3s)


def _deg_finish_body(hist_ref, dinv_ref, dinv2_ref):
    deg = jnp.sum(hist_ref[...], axis=0) + 1.0
    dinv_ref[...] = lax.rsqrt(deg)
    dinv2_ref[...] = 1.0 / deg


def _deg_finish_call(hists):
    return pl.pallas_call(
        _deg_finish_body,
        out_shape=[
            jax.ShapeDtypeStruct((NACC,), _f32),
            jax.ShapeDtypeStruct((NACC,), _f32),
        ],
    )(hists)


def _row_spec():
    return pl.BlockSpec((BLK, NH), lambda i: (i, 0))


def _col1_spec():
    return pl.BlockSpec((BLK, 1), lambda i: (i, 0))


def _w_spec():
    return pl.BlockSpec((NH, NH), lambda i: (0, 0))


def _b_spec():
    return pl.BlockSpec((NH,), lambda i: (0,))


def _p_spec():
    return pl.BlockSpec((NC, BLK, NH), lambda i: (0, i, 0))


def _in_mlp_body(x_ref, w_ref, b_ref, o_ref):
    t = jnp.dot(x_ref[...], w_ref[...], preferred_element_type=_f32)
    o_ref[...] = jnp.maximum(t + b_ref[...], 0.0)


def _in_mlp_call(x, W_in, b_in):
    return pl.pallas_call(
        _in_mlp_body,
        grid=(GRID,),
        in_specs=[_row_spec(), _w_spec(), _b_spec()],
        out_specs=_row_spec(),
        out_shape=jax.ShapeDtypeStruct((N, NH), _f32),
    )(x, W_in, b_in)


def _layer_pre_body(h_ref, wf_ref, dinv_ref, y0_ref, z_ref):
    y0 = jax.nn.sigmoid(
        jnp.dot(h_ref[...], wf_ref[...], preferred_element_type=_f32))
    y0_ref[...] = y0
    z_ref[...] = dinv_ref[...] * y0


def _layer_pre_call(h, Wf, dinv):
    return pl.pallas_call(
        _layer_pre_body,
        grid=(GRID,),
        in_specs=[_row_spec(), _w_spec(), _col1_spec()],
        out_specs=[_row_spec(), _row_spec()],
        out_shape=[
            jax.ShapeDtypeStruct((N, NH), _f32),
            jax.ShapeDtypeStruct((N, NH), _f32),
        ],
    )(h, Wf, dinv)


def _combine0_body(p_ref, yin_ref, y0_ref, wa_ref, dinv_ref, dinv2_ref,
                   y_ref, z_ref):
    sv = dinv_ref[...] * (p_ref[0] + p_ref[1]) + dinv2_ref[...] * yin_ref[...]
    t = jnp.dot(sv, wa_ref[...], preferred_element_type=_f32)
    y0 = y0_ref[...]
    y = y0 * y0 * t
    y_ref[...] = y
    z_ref[...] = dinv_ref[...] * y


def _combine0_call(p, yin, y0, Wa, dinv, dinv2):
    return pl.pallas_call(
        _combine0_body,
        grid=(GRID,),
        in_specs=[_p_spec(), _row_spec(), _row_spec(), _w_spec(),
                  _col1_spec(), _col1_spec()],
        out_specs=[_row_spec(), _row_spec()],
        out_shape=[
            jax.ShapeDtypeStruct((N, NH), _f32),
            jax.ShapeDtypeStruct((N, NH), _f32),
        ],
    )(p, yin, y0, Wa, dinv, dinv2)


def _combine1_mid_body(p_ref, yin_ref, y0_ref, wa_ref, dinv_ref, dinv2_ref,
                       prev_ref, h_ref):
    sv = dinv_ref[...] * (p_ref[0] + p_ref[1]) + dinv2_ref[...] * yin_ref[...]
    t = jnp.dot(sv, wa_ref[...], preferred_element_type=_f32)
    y = y0_ref[...] * t
    h_ref[...] = jnp.maximum(y, 0.0) + prev_ref[...]


def _combine1_mid_call(p, yin, y0, Wa, dinv, dinv2, prev):
    return pl.pallas_call(
        _combine1_mid_body,
        grid=(GRID,),
        in_specs=[_p_spec(), _row_spec(), _row_spec(), _w_spec(),
                  _col1_spec(), _col1_spec(), _row_spec()],
        out_specs=_row_spec(),
        out_shape=jax.ShapeDtypeStruct((N, NH), _f32),
    )(p, yin, y0, Wa, dinv, dinv2, prev)


def _combine1_last_body(p_ref, yin_ref, y0_ref, wa_ref, dinv_ref, dinv2_ref,
                        y_ref):
    sv = dinv_ref[...] * (p_ref[0] + p_ref[1]) + dinv2_ref[...] * yin_ref[...]
    t = jnp.dot(sv, wa_ref[...], preferred_element_type=_f32)
    y_ref[...] = y0_ref[...] * t


def _combine1_last_call(p, yin, y0, Wa, dinv, dinv2):
    return pl.pallas_call(
        _combine1_last_body,
        grid=(GRID,),
        in_specs=[_p_spec(), _row_spec(), _row_spec(), _w_spec(),
                  _col1_spec(), _col1_spec()],
        out_specs=_row_spec(),
        out_shape=jax.ShapeDtypeStruct((N, NH), _f32),
    )(p, yin, y0, Wa, dinv, dinv2)


def _out_mlp_body(y_ref, w1_ref, b1_ref, w2_ref, b2_ref, o_ref):
    t = jnp.dot(y_ref[...], w1_ref[...], preferred_element_type=_f32)
    t = jnp.maximum(t + b1_ref[...], 0.0)
    o_ref[...] = jnp.dot(t, w2_ref[...],
                         preferred_element_type=_f32) + b2_ref[...]


def _out_mlp_call(y, Wo1, bo1, Wo2, bo2):
    return pl.pallas_call(
        _out_mlp_body,
        grid=(GRID,),
        in_specs=[_row_spec(), _w_spec(), _b_spec(), _w_spec(), _b_spec()],
        out_specs=_row_spec(),
        out_shape=jax.ShapeDtypeStruct((N, NOUT), _f32),
    )(y, Wo1, bo1, Wo2, bo2)


def kernel(x, edge_index, edge_attr, W_in, b_in, Wf0, Wa0, Wf1, Wa1, Wf2, Wa2,
           Wo1, bo1, Wo2, bo2):
    ei = edge_index.astype(_i32)
    eip = jnp.concatenate([ei, jnp.zeros((2, EPAD - E), _i32)], axis=1)
    row3d = eip[0].reshape(NW, NCHD, CD)
    col3d = eip[1].reshape(NW, NCHD, CD)

    hists, re3d = _deg_call(row3d, col3d)
    dinv_flat, dinv2_flat = _deg_finish_call(hists.reshape(NW, NACC))
    dinv = dinv_flat.reshape(NACC, 1)
    dinv2 = dinv2_flat.reshape(NACC, 1)

    col3s = col3d.reshape(NW, NG, CH)
    re3s = re3d.reshape(NW, NG, CH)

    h = _in_mlp_call(x, W_in, b_in)
    prev = h
    y = h
    for i, (Wf, Wa) in enumerate(((Wf0, Wa0), (Wf1, Wa1), (Wf2, Wa2))):
        y0, z = _layer_pre_call(h, Wf, dinv)
        p = _spmm_call(z, col3s, re3s)
        y, z2 = _combine0_call(p, y0, y0, Wa, dinv, dinv2)
        p = _spmm_call(z2, col3s, re3s)
        if i < 2:
            h = _combine1_mid_call(p, y, y0, Wa, dinv, dinv2, prev)
            prev = h
        else:
            y = _combine1_last_call(p, y, y0, Wa, dinv, dinv2)
    return _out_mlp_call(y, Wo1, bo1, Wo2, bo2)


# speedup vs baseline: 1.0022x; 1.0022x over previous
import jax
import jax.numpy as jnp
from jax import lax
from jax.experimental import pallas as pl
from jax.experimental.pallas import tpu as pltpu
from jax.experimental.pallas import tpu_sc as plsc

N = 10000
E = 320000
NH = 128
NOUT = 128

NC = 2
NS = 16
NW = NC * NS
EPAD = 327680
EPW = EPAD // NW
CD = 80
NCHD = EPW // CD
CH = 40
NG = EPW // CH
NB = NG // 8
NACC = 10112
RPT = NACC // NS
DUMMY = N
BLK = 2048
GRID = 5

_f32 = jnp.float32
_i32 = jnp.int32


def _mesh():
    return plsc.VectorSubcoreMesh(core_axis_name="c", subcore_axis_name="s")


def _deg_body(row_hbm, col_hbm, hist_hbm, re_hbm, rowb, colb, reb, hist):
    c = lax.axis_index("c")
    s = lax.axis_index("s")
    wid = c * NS + s
    pltpu.sync_copy(row_hbm.at[wid], rowb)
    pltpu.sync_copy(col_hbm.at[wid], colb)

    def zero(i, _):
        hist[pl.ds(i * 16, 16)] = jnp.zeros((16,), _f32)
        return 0

    lax.fori_loop(0, NACC // 16, zero, 0)

    ones = jnp.ones((16,), _f32)

    def outer(i, _):
        def inner(k, _):
            rv = rowb[i, pl.ds(k * 16, 16)]
            cv = colb[i, pl.ds(k * 16, 16)]
            m = rv != cv
            plsc.addupdate_scatter(hist, [cv], ones, mask=m)
            reb[i, pl.ds(k * 16, 16)] = jnp.where(m, rv, DUMMY)
            return 0

        lax.fori_loop(0, CD // 16, inner, 0)
        return 0

    lax.fori_loop(0, NCHD, outer, 0)
    pltpu.sync_copy(hist, hist_hbm.at[pl.ds(wid * NACC, NACC)])
    pltpu.sync_copy(reb, re_hbm.at[wid])


def _deg_call(row3d, col3d):
    kern = pl.kernel(
        _deg_body,
        out_type=[
            jax.ShapeDtypeStruct((NW * NACC,), _f32),
            jax.ShapeDtypeStruct((NW, NCHD, CD), _i32),
        ],
        mesh=_mesh(),
        scratch_types=[
            pltpu.VMEM((NCHD, CD), _i32),
            pltpu.VMEM((NCHD, CD), _i32),
            pltpu.VMEM((NCHD, CD), _i32),
            pltpu.VMEM((NACC,), _f32),
        ],
        compiler_params=pltpu.CompilerParams(needs_layout_passes=False),
    )
    return kern(row3d, col3d)


def _spmm_body(z_hbm, col_hbm, re_hbm, out_hbm, colring, rering, dbuf, acc,
               *sems):
    gsems = sems[0:8]
    ssems = sems[8:16]
    icsems = sems[16:20]
    irsems = sems[20:24]
    c = lax.axis_index("c")
    s = lax.axis_index("s")
    wid = c * NS + s

    slots = [dbuf.at[pl.ds(j * CH, CH)] for j in range(8)]

    def ifire(b, k):
        pltpu.async_copy(col_hbm.at[wid, pl.ds(8 * b, 8)],
                         colring.at[pl.ds(k * 8, 8)], icsems[k])
        pltpu.async_copy(re_hbm.at[wid, pl.ds(8 * b, 8)],
                         rering.at[pl.ds(k * 8, 8)], irsems[k])

    def iwait(b, k):
        pltpu.make_async_copy(col_hbm.at[wid, pl.ds(8 * b, 8)],
                              colring.at[pl.ds(k * 8, 8)],
                              icsems[k]).wait()
        pltpu.make_async_copy(re_hbm.at[wid, pl.ds(8 * b, 8)],
                              rering.at[pl.ds(k * 8, 8)],
                              irsems[k]).wait()

    def gfire(r, j):
        pltpu.async_copy(z_hbm.at[colring.at[r]], slots[j], gsems[j])

    def gwait(r, j):
        pltpu.make_async_copy(z_hbm.at[colring.at[r]], slots[j],
                              gsems[j]).wait()

    def sfire(r, j):
        pltpu.async_copy(slots[j], acc.at[rering.at[r]], ssems[j], add=True)

    def swait(r, j):
        pltpu.make_async_copy(slots[j], acc.at[rering.at[r]],
                              ssems[j]).wait()

    def zrow(i, _):
        for k in range(8):
            dbuf[i, pl.ds(k * 16, 16)] = jnp.zeros((16,), _f32)
        return 0

    lax.fori_loop(0, CH, zrow, 0)

    def zacc(j, _):
        pltpu.sync_copy(slots[0], acc.at[pl.ds(s * RPT + j * CH, CH)])
        return 0

    lax.fori_loop(0, RPT // CH, zacc, 0)
    pltpu.sync_copy(dbuf.at[pl.ds(0, RPT - (RPT // CH) * CH)],
                    acc.at[pl.ds(s * RPT + (RPT // CH) * CH,
                                 RPT - (RPT // CH) * CH)])
    plsc.subcore_barrier()

    ifire(0, 0)
    ifire(1, 1)
    ifire(2, 2)
    iwait(0, 0)
    for j in range(3):
        gfire(j, j)

    def body(p, _):
        pe = lax.rem(p, 2) == 0

        def row(g):
            return lax.rem(g, 32)

        for j in range(16):
            g = 16 * p + j
            gwait(row(g), j % 8)
            sfire(row(g), j % 8)
            if j < 5:
                @pl.when(p > 0)
                def _():
                    swait(row(g - 5), (j - 5) % 8)
            else:
                swait(row(g - 5), (j - 5) % 8)
            if j == 5:
                @pl.when(pe)
                def _():
                    iwait(2 * p + 1, 1)

                @pl.when(jnp.logical_not(pe))
                def _():
                    iwait(2 * p + 1, 3)
            if j == 13:
                @pl.when(jnp.logical_and(pe, 2 * p + 2 < NB))
                def _():
                    iwait(2 * p + 2, 2)

                @pl.when(jnp.logical_and(jnp.logical_not(pe),
                                         2 * p + 2 < NB))
                def _():
                    iwait(2 * p + 2, 0)
            if j == 3:
                @pl.when(jnp.logical_and(pe, 2 * p + 3 < NB))
                def _():
                    ifire(2 * p + 3, 3)

                @pl.when(jnp.logical_and(jnp.logical_not(pe),
                                         2 * p + 3 < NB))
                def _():
                    ifire(2 * p + 3, 1)
            if j == 11:
                @pl.when(jnp.logical_and(pe, 2 * p + 4 < NB))
                def _():
                    ifire(2 * p + 4, 0)

                @pl.when(jnp.logical_and(jnp.logical_not(pe),
                                         2 * p + 4 < NB))
                def _():
                    ifire(2 * p + 4, 2)
            if j < 13:
                gfire(row(g + 3), (j + 3) % 8)
            else:
                @pl.when(g + 3 < NG)
                def _():
                    gfire(row(g + 3), (j + 3) % 8)
        return 0

    lax.fori_loop(0, NG // 16, body, 0)
    swait(27, 3)
    swait(28, 4)
    swait(29, 5)
    swait(30, 6)
    swait(31, 7)

    plsc.subcore_barrier()
    pltpu.sync_copy(acc.at[pl.ds(s * RPT, RPT)],
                    out_hbm.at[c, pl.ds(s * RPT, RPT)])


def _spmm_call(z, col3s, re3s):
    kern = pl.kernel(
        _spmm_body,
        out_type=jax.ShapeDtypeStruct((NC, NACC, NH), _f32),
        mesh=_mesh(),
        scratch_types=[
            pltpu.VMEM((32, CH), _i32),
            pltpu.VMEM((32, CH), _i32),
            pltpu.VMEM((8 * CH, NH), _f32),
            pltpu.VMEM_SHARED((NACC, NH), _f32),
        ] + [pltpu.SemaphoreType.DMA] * 24,
        compiler_params=pltpu.CompilerParams(needs_layout_passes=False),
    )
    return kern(z, col3s, re3s)


def _deg_finish_body(hist_ref, dinv_ref, dinv2_ref):
    deg = jnp.sum(hist_ref[...], axis=0) + 1.0
    dinv_ref[...] = lax.rsqrt(deg)
    dinv2_ref[...] = 1.0 / deg


def _deg_finish_call(hists):
    return pl.pallas_call(
        _deg_finish_body,
        out_shape=[
            jax.ShapeDtypeStruct((NACC,), _f32),
            jax.ShapeDtypeStruct((NACC,), _f32),
        ],
    )(hists)


def _row_spec():
    return pl.BlockSpec((BLK, NH), lambda i: (i, 0))


def _col1_spec():
    return pl.BlockSpec((BLK, 1), lambda i: (i, 0))


def _w_spec():
    return pl.BlockSpec((NH, NH), lambda i: (0, 0))


def _b_spec():
    return pl.BlockSpec((NH,), lambda i: (0,))


def _p_spec():
    return pl.BlockSpec((NC, BLK, NH), lambda i: (0, i, 0))


def _in_mlp_body(x_ref, w_ref, b_ref, o_ref):
    t = jnp.dot(x_ref[...], w_ref[...], preferred_element_type=_f32)
    o_ref[...] = jnp.maximum(t + b_ref[...], 0.0)


def _in_mlp_call(x, W_in, b_in):
    return pl.pallas_call(
        _in_mlp_body,
        grid=(GRID,),
        in_specs=[_row_spec(), _w_spec(), _b_spec()],
        out_specs=_row_spec(),
        out_shape=jax.ShapeDtypeStruct((N, NH), _f32),
    )(x, W_in, b_in)


def _layer_pre_body(h_ref, wf_ref, dinv_ref, y0_ref, z_ref):
    y0 = jax.nn.sigmoid(
        jnp.dot(h_ref[...], wf_ref[...], preferred_element_type=_f32))
    y0_ref[...] = y0
    z_ref[...] = dinv_ref[...] * y0


def _layer_pre_call(h, Wf, dinv):
    return pl.pallas_call(
        _layer_pre_body,
        grid=(GRID,),
        in_specs=[_row_spec(), _w_spec(), _col1_spec()],
        out_specs=[_row_spec(), _row_spec()],
        out_shape=[
            jax.ShapeDtypeStruct((N, NH), _f32),
            jax.ShapeDtypeStruct((N, NH), _f32),
        ],
    )(h, Wf, dinv)


def _combine0_body(p_ref, yin_ref, y0_ref, wa_ref, dinv_ref, dinv2_ref,
                   y_ref, z_ref):
    sv = dinv_ref[...] * (p_ref[0] + p_ref[1]) + dinv2_ref[...] * yin_ref[...]
    t = jnp.dot(sv, wa_ref[...], preferred_element_type=_f32)
    y0 = y0_ref[...]
    y = y0 * y0 * t
    y_ref[...] = y
    z_ref[...] = dinv_ref[...] * y


def _combine0_call(p, yin, y0, Wa, dinv, dinv2):
    return pl.pallas_call(
        _combine0_body,
        grid=(GRID,),
        in_specs=[_p_spec(), _row_spec(), _row_spec(), _w_spec(),
                  _col1_spec(), _col1_spec()],
        out_specs=[_row_spec(), _row_spec()],
        out_shape=[
            jax.ShapeDtypeStruct((N, NH), _f32),
            jax.ShapeDtypeStruct((N, NH), _f32),
        ],
    )(p, yin, y0, Wa, dinv, dinv2)


def _combine1_mid_body(p_ref, yin_ref, y0_ref, wa_ref, dinv_ref, dinv2_ref,
                       prev_ref, h_ref):
    sv = dinv_ref[...] * (p_ref[0] + p_ref[1]) + dinv2_ref[...] * yin_ref[...]
    t = jnp.dot(sv, wa_ref[...], preferred_element_type=_f32)
    y = y0_ref[...] * t
    h_ref[...] = jnp.maximum(y, 0.0) + prev_ref[...]


def _combine1_mid_call(p, yin, y0, Wa, dinv, dinv2, prev):
    return pl.pallas_call(
        _combine1_mid_body,
        grid=(GRID,),
        in_specs=[_p_spec(), _row_spec(), _row_spec(), _w_spec(),
                  _col1_spec(), _col1_spec(), _row_spec()],
        out_specs=_row_spec(),
        out_shape=jax.ShapeDtypeStruct((N, NH), _f32),
    )(p, yin, y0, Wa, dinv, dinv2, prev)


def _combine1_last_body(p_ref, yin_ref, y0_ref, wa_ref, dinv_ref, dinv2_ref,
                        y_ref):
    sv = dinv_ref[...] * (p_ref[0] + p_ref[1]) + dinv2_ref[...] * yin_ref[...]
    t = jnp.dot(sv, wa_ref[...], preferred_element_type=_f32)
    y_ref[...] = y0_ref[...] * t


def _combine1_last_call(p, yin, y0, Wa, dinv, dinv2):
    return pl.pallas_call(
        _combine1_last_body,
        grid=(GRID,),
        in_specs=[_p_spec(), _row_spec(), _row_spec(), _w_spec(),
                  _col1_spec(), _col1_spec()],
        out_specs=_row_spec(),
        out_shape=jax.ShapeDtypeStruct((N, NH), _f32),
    )(p, yin, y0, Wa, dinv, dinv2)


def _out_mlp_body(y_ref, w1_ref, b1_ref, w2_ref, b2_ref, o_ref):
    t = jnp.dot(y_ref[...], w1_ref[...], preferred_element_type=_f32)
    t = jnp.maximum(t + b1_ref[...], 0.0)
    o_ref[...] = jnp.dot(t, w2_ref[...],
                         preferred_element_type=_f32) + b2_ref[...]


def _out_mlp_call(y, Wo1, bo1, Wo2, bo2):
    return pl.pallas_call(
        _out_mlp_body,
        grid=(GRID,),
        in_specs=[_row_spec(), _w_spec(), _b_spec(), _w_spec(), _b_spec()],
        out_specs=_row_spec(),
        out_shape=jax.ShapeDtypeStruct((N, NOUT), _f32),
    )(y, Wo1, bo1, Wo2, bo2)


def kernel(x, edge_index, edge_attr, W_in, b_in, Wf0, Wa0, Wf1, Wa1, Wf2, Wa2,
           Wo1, bo1, Wo2, bo2):
    ei = edge_index.astype(_i32)
    eip = jnp.concatenate([ei, jnp.zeros((2, EPAD - E), _i32)], axis=1)
    row3d = eip[0].reshape(NW, NCHD, CD)
    col3d = eip[1].reshape(NW, NCHD, CD)

    hists, re3d = _deg_call(row3d, col3d)
    dinv_flat, dinv2_flat = _deg_finish_call(hists.reshape(NW, NACC))
    dinv = dinv_flat.reshape(NACC, 1)
    dinv2 = dinv2_flat.reshape(NACC, 1)

    col3s = col3d.reshape(NW, NG, CH)
    re3s = re3d.reshape(NW, NG, CH)

    h = _in_mlp_call(x, W_in, b_in)
    prev = h
    y = h
    for i, (Wf, Wa) in enumerate(((Wf0, Wa0), (Wf1, Wa1), (Wf2, Wa2))):
        y0, z = _layer_pre_call(h, Wf, dinv)
        p = _spmm_call(z, col3s, re3s)
        y, z2 = _combine0_call(p, y0, y0, Wa, dinv, dinv2)
        p = _spmm_call(z2, col3s, re3s)
        if i < 2:
            h = _combine1_mid_call(p, y, y0, Wa, dinv, dinv2, prev)
            prev = h
        else:
            y = _combine1_last_call(p, y, y0, Wa, dinv, dinv2)
    return _out_mlp_call(y, Wo1, bo1, Wo2, bo2)


# speedup vs baseline: 3.0254x; 3.0187x over previous
import functools

import jax
import jax.numpy as jnp
from jax import lax
from jax.experimental import pallas as pl
from jax.experimental.pallas import tpu as pltpu
from jax.experimental.pallas import tpu_sc as plsc

N = 10000
E = 320000
NH = 128
NOUT = 128

NC = 2
NS = 16
NW = NC * NS
EPW = E // NW
C = 80
NCHUNK = EPW // C
ROWS2D = E // C
NACC = 10112
RPT = NACC // NS
DUMMY = N
BLK = 2048
GRID = 5

_f32 = jnp.float32
_i32 = jnp.int32


def _mesh():
    return plsc.VectorSubcoreMesh(core_axis_name="c", subcore_axis_name="s")


def _deg_body(row_hbm, col_hbm, hist_hbm, re_hbm, rowb, colb, reb, hist):
    c = lax.axis_index("c")
    s = lax.axis_index("s")
    wid = c * NS + s
    pltpu.sync_copy(row_hbm.at[wid], rowb)
    pltpu.sync_copy(col_hbm.at[wid], colb)

    def zero(i, _):
        hist[pl.ds(i * 16, 16)] = jnp.zeros((16,), _f32)
        return 0

    lax.fori_loop(0, NACC // 16, zero, 0)

    ones = jnp.ones((16,), _f32)

    def outer(i, _):
        def inner(k, _):
            rv = rowb[i, pl.ds(k * 16, 16)]
            cv = colb[i, pl.ds(k * 16, 16)]
            m = rv != cv
            plsc.addupdate_scatter(hist, [cv], ones, mask=m)
            reb[i, pl.ds(k * 16, 16)] = jnp.where(m, rv, DUMMY)
            return 0

        lax.fori_loop(0, C // 16, inner, 0)
        return 0

    lax.fori_loop(0, NCHUNK, outer, 0)
    pltpu.sync_copy(hist, hist_hbm.at[pl.ds(wid * NACC, NACC)])
    pltpu.sync_copy(reb, re_hbm.at[wid])


def _deg_call(row3d, col3d):
    kern = pl.kernel(
        _deg_body,
        out_type=[
            jax.ShapeDtypeStruct((NW * NACC,), _f32),
            jax.ShapeDtypeStruct((NW, NCHUNK, C), _i32),
        ],
        mesh=_mesh(),
        scratch_types=[
            pltpu.VMEM((NCHUNK, C), _i32),
            pltpu.VMEM((NCHUNK, C), _i32),
            pltpu.VMEM((NCHUNK, C), _i32),
            pltpu.VMEM((NACC,), _f32),
        ],
        compiler_params=pltpu.CompilerParams(needs_layout_passes=False),
    )
    return kern(row3d, col3d)


def _spmm_body(z_hbm, col_hbm, re_hbm, out_hbm, colb, reb, dbA, dbB, acc,
               semA, semB, gsemC, gsemD, ssemA, ssemB):
    c = lax.axis_index("c")
    s = lax.axis_index("s")
    wid = c * NS + s
    pltpu.sync_copy(col_hbm.at[pl.ds(wid * EPW, EPW)], colb)
    pltpu.sync_copy(re_hbm.at[wid], reb)

    def zrow(i, _):
        for k in range(8):
            dbA[i, pl.ds(k * 16, 16)] = jnp.zeros((16,), _f32)
        return 0

    lax.fori_loop(0, C, zrow, 0)

    def zacc(j, _):
        pltpu.sync_copy(dbA, acc.at[pl.ds(s * RPT + j * C, C)])
        return 0

    lax.fori_loop(0, RPT // C, zacc, 0)
    pltpu.sync_copy(dbA.at[pl.ds(0, RPT - (RPT // C) * C)],
                    acc.at[pl.ds(s * RPT + (RPT // C) * C,
                                 RPT - (RPT // C) * C)])
    plsc.subcore_barrier()

    CH = C // 2
    NG = NCHUNK * 2

    def cidx(g):
        return colb.at[pl.ds(g * CH, CH)]

    slots = [dbA.at[pl.ds(0, CH)], dbA.at[pl.ds(CH, CH)],
             dbB.at[pl.ds(0, CH)], dbB.at[pl.ds(CH, CH)]]
    gsems = [semA, semB, gsemC, gsemD]

    def gfire(g, j):
        pltpu.async_copy(z_hbm.at[cidx(g)], slots[j], gsems[j])

    def gwait(g, j):
        pltpu.make_async_copy(z_hbm.at[cidx(g)], slots[j], gsems[j]).wait()

    def swaitA(h):
        pltpu.make_async_copy(dbA, acc.at[reb.at[h]], ssemA).wait()

    def swaitB(h):
        pltpu.make_async_copy(dbB, acc.at[reb.at[h]], ssemB).wait()

    for j in range(4):
        gfire(j, j)

    def pair(p, _):
        g = 4 * p
        h0 = 2 * p
        h1 = h0 + 1
        gwait(g, 0)
        gwait(g + 1, 1)
        pltpu.async_copy(dbA, acc.at[reb.at[h0]], ssemA, add=True)
        gwait(g + 2, 2)
        gwait(g + 3, 3)
        pltpu.async_copy(dbB, acc.at[reb.at[h1]], ssemB, add=True)
        swaitA(h0)

        @pl.when(g + 4 < NG)
        def _():
            gfire(g + 4, 0)

        @pl.when(g + 5 < NG)
        def _():
            gfire(g + 5, 1)

        swaitB(h1)

        @pl.when(g + 6 < NG)
        def _():
            gfire(g + 6, 2)

        @pl.when(g + 7 < NG)
        def _():
            gfire(g + 7, 3)

        return 0

    lax.fori_loop(0, NCHUNK // 2, pair, 0)
    gwait(NG - 2, 0)
    gwait(NG - 1, 1)
    pltpu.async_copy(dbA, acc.at[reb.at[NCHUNK - 1]], ssemA, add=True)
    swaitA(NCHUNK - 1)

    plsc.subcore_barrier()
    pltpu.sync_copy(acc.at[pl.ds(s * RPT, RPT)],
                    out_hbm.at[c, pl.ds(s * RPT, RPT)])


def _spmm_call(z, col_flat, re3d):
    kern = pl.kernel(
        _spmm_body,
        out_type=jax.ShapeDtypeStruct((NC, NACC, NH), _f32),
        mesh=_mesh(),
        scratch_types=[
            pltpu.VMEM((EPW,), _i32),
            pltpu.VMEM((NCHUNK, C), _i32),
            pltpu.VMEM((C, NH), _f32),
            pltpu.VMEM((C, NH), _f32),
            pltpu.VMEM_SHARED((NACC, NH), _f32),
            pltpu.SemaphoreType.DMA,
            pltpu.SemaphoreType.DMA,
            pltpu.SemaphoreType.DMA,
            pltpu.SemaphoreType.DMA,
            pltpu.SemaphoreType.DMA,
            pltpu.SemaphoreType.DMA,
        ],
        compiler_params=pltpu.CompilerParams(needs_layout_passes=False),
    )
    return kern(z, col_flat, re3d)


def _deg_finish_body(hist_ref, dinv_ref, dinv2_ref):
    deg = jnp.sum(hist_ref[...], axis=0) + 1.0
    dinv_ref[...] = lax.rsqrt(deg)
    dinv2_ref[...] = 1.0 / deg


def _deg_finish_call(hists):
    return pl.pallas_call(
        _deg_finish_body,
        out_shape=[
            jax.ShapeDtypeStruct((NACC,), _f32),
            jax.ShapeDtypeStruct((NACC,), _f32),
        ],
    )(hists)


def _row_spec():
    return pl.BlockSpec((BLK, NH), lambda i: (i, 0))


def _col1_spec():
    return pl.BlockSpec((BLK, 1), lambda i: (i, 0))


def _w_spec():
    return pl.BlockSpec((NH, NH), lambda i: (0, 0))


def _b_spec():
    return pl.BlockSpec((NH,), lambda i: (0,))


def _p_spec():
    return pl.BlockSpec((NC, BLK, NH), lambda i: (0, i, 0))


def _in_pre_body(x_ref, w_ref, b_ref, wf_ref, dinv_ref, h_ref, y0_ref, z_ref):
    t = jnp.dot(x_ref[...], w_ref[...], preferred_element_type=_f32)
    h = jnp.maximum(t + b_ref[...], 0.0)
    h_ref[...] = h
    y0 = jax.nn.sigmoid(jnp.dot(h, wf_ref[...], preferred_element_type=_f32))
    y0_ref[...] = y0
    z_ref[...] = dinv_ref[...] * y0


def _in_pre_call(x, W_in, b_in, Wf, dinv):
    return pl.pallas_call(
        _in_pre_body,
        grid=(GRID,),
        in_specs=[_row_spec(), _w_spec(), _b_spec(), _w_spec(), _col1_spec()],
        out_specs=[_row_spec(), _row_spec(), _row_spec()],
        out_shape=[
            jax.ShapeDtypeStruct((N, NH), _f32),
            jax.ShapeDtypeStruct((N, NH), _f32),
            jax.ShapeDtypeStruct((N, NH), _f32),
        ],
    )(x, W_in, b_in, Wf, dinv)


def _combine0_body(p_ref, yin_ref, y0_ref, wa_ref, dinv_ref, dinv2_ref,
                   y_ref, z_ref):
    sv = dinv_ref[...] * (p_ref[0] + p_ref[1]) + dinv2_ref[...] * yin_ref[...]
    t = jnp.dot(sv, wa_ref[...], preferred_element_type=_f32)
    y0 = y0_ref[...]
    y = y0 * y0 * t
    y_ref[...] = y
    z_ref[...] = dinv_ref[...] * y


def _combine0_call(p, yin, y0, Wa, dinv, dinv2):
    return pl.pallas_call(
        _combine0_body,
        grid=(GRID,),
        in_specs=[_p_spec(), _row_spec(), _row_spec(), _w_spec(),
                  _col1_spec(), _col1_spec()],
        out_specs=[_row_spec(), _row_spec()],
        out_shape=[
            jax.ShapeDtypeStruct((N, NH), _f32),
            jax.ShapeDtypeStruct((N, NH), _f32),
        ],
    )(p, yin, y0, Wa, dinv, dinv2)


def _combine1_pre_body(p_ref, yin_ref, y0_ref, wa_ref, dinv_ref, dinv2_ref,
                       prev_ref, wf_ref, h_ref, y0n_ref, zn_ref):
    sv = dinv_ref[...] * (p_ref[0] + p_ref[1]) + dinv2_ref[...] * yin_ref[...]
    t = jnp.dot(sv, wa_ref[...], preferred_element_type=_f32)
    y = y0_ref[...] * t
    h = jnp.maximum(y, 0.0) + prev_ref[...]
    h_ref[...] = h
    y0n = jax.nn.sigmoid(jnp.dot(h, wf_ref[...], preferred_element_type=_f32))
    y0n_ref[...] = y0n
    zn_ref[...] = dinv_ref[...] * y0n


def _combine1_pre_call(p, yin, y0, Wa, dinv, dinv2, prev, Wf_next):
    return pl.pallas_call(
        _combine1_pre_body,
        grid=(GRID,),
        in_specs=[_p_spec(), _row_spec(), _row_spec(), _w_spec(),
                  _col1_spec(), _col1_spec(), _row_spec(), _w_spec()],
        out_specs=[_row_spec(), _row_spec(), _row_spec()],
        out_shape=[
            jax.ShapeDtypeStruct((N, NH), _f32),
            jax.ShapeDtypeStruct((N, NH), _f32),
            jax.ShapeDtypeStruct((N, NH), _f32),
        ],
    )(p, yin, y0, Wa, dinv, dinv2, prev, Wf_next)


def _combine1_last_body(p_ref, yin_ref, y0_ref, wa_ref, dinv_ref, dinv2_ref,
                        y_ref):
    sv = dinv_ref[...] * (p_ref[0] + p_ref[1]) + dinv2_ref[...] * yin_ref[...]
    t = jnp.dot(sv, wa_ref[...], preferred_element_type=_f32)
    y_ref[...] = y0_ref[...] * t


def _combine1_last_call(p, yin, y0, Wa, dinv, dinv2):
    return pl.pallas_call(
        _combine1_last_body,
        grid=(GRID,),
        in_specs=[_p_spec(), _row_spec(), _row_spec(), _w_spec(),
                  _col1_spec(), _col1_spec()],
        out_specs=_row_spec(),
        out_shape=jax.ShapeDtypeStruct((N, NH), _f32),
    )(p, yin, y0, Wa, dinv, dinv2)


def _out_mlp_body(y_ref, w1_ref, b1_ref, w2_ref, b2_ref, o_ref):
    t = jnp.dot(y_ref[...], w1_ref[...], preferred_element_type=_f32)
    t = jnp.maximum(t + b1_ref[...], 0.0)
    o_ref[...] = jnp.dot(t, w2_ref[...],
                         preferred_element_type=_f32) + b2_ref[...]


def _out_mlp_call(y, Wo1, bo1, Wo2, bo2):
    return pl.pallas_call(
        _out_mlp_body,
        grid=(GRID,),
        in_specs=[_row_spec(), _w_spec(), _b_spec(), _w_spec(), _b_spec()],
        out_specs=_row_spec(),
        out_shape=jax.ShapeDtypeStruct((N, NOUT), _f32),
    )(y, Wo1, bo1, Wo2, bo2)


def kernel(x, edge_index, edge_attr, W_in, b_in, Wf0, Wa0, Wf1, Wa1, Wf2, Wa2,
           Wo1, bo1, Wo2, bo2):
    ei = edge_index.astype(_i32)
    row3d = ei[0].reshape(NW, NCHUNK, C)
    col3d = ei[1].reshape(NW, NCHUNK, C)
    col_flat = ei[1]

    hists, re3d = _deg_call(row3d, col3d)
    dinv_flat, dinv2_flat = _deg_finish_call(hists.reshape(NW, NACC))
    dinv = dinv_flat.reshape(NACC, 1)
    dinv2 = dinv2_flat.reshape(NACC, 1)

    h, y0, z = _in_pre_call(x, W_in, b_in, Wf0, dinv)
    prev = h
    Wfs = (Wf0, Wf1, Wf2)
    Was = (Wa0, Wa1, Wa2)
    for i in range(3):
        p = _spmm_call(z, col_flat, re3d)
        y, z2 = _combine0_call(p, y0, y0, Was[i], dinv, dinv2)
        p = _spmm_call(z2, col_flat, re3d)
        if i < 2:
            h, y0, z = _combine1_pre_call(p, y, y0, Was[i], dinv, dinv2,
                                          prev, Wfs[i + 1])
            prev = h
        else:
            y = _combine1_last_call(p, y, y0, Was[i], dinv, dinv2)
    return _out_mlp_call(y, Wo1, bo1, Wo2, bo2)
